# baseline (speedup 1.0000x reference)
import functools

import jax
import jax.numpy as jnp
from jax import lax
from jax.experimental import pallas as pl
from jax.experimental.pallas import tpu as pltpu
from jax.experimental.pallas import tpu_sc as plsc

N_NODES = 10000
N_EDGES = 320000
D_FEAT = 128
N_CLASSES = 64

NC, NS = 2, 16
NW = NC * NS
CHUNK = 128
NCH = 80
E_PK = NW * NCH * CHUNK
PADE = E_PK - N_EDGES
N_PAD = 10240
RPT = N_PAD // NS
DEPTH = 5
NG = NCH // DEPTH


def _mm_body(feat_ref, w_ref, o_ref):
    o_ref[:N_NODES] = lax.dot_general(
        feat_ref[...], w_ref[...],
        (((1,), (1,)), ((), ())),
        preferred_element_type=jnp.float32,
    )
    o_ref[N_NODES:] = jnp.zeros((N_PAD - N_NODES, N_CLASSES), jnp.float32)


def _tc_matmul(feat, W):
    return pl.pallas_call(
        _mm_body,
        out_shape=jax.ShapeDtypeStruct((N_PAD, N_CLASSES), jnp.float32),
    )(feat, W)


def _mid_body(p_ref, o_ref):
    o_ref[...] = p_ref[0] + p_ref[1]


def _tc_mid(p):
    return pl.pallas_call(
        _mid_body,
        out_shape=jax.ShapeDtypeStruct((N_PAD, N_CLASSES), jnp.float32),
    )(p)


def _comb_body(q_ref, b_ref, o_ref):
    o_ref[...] = (q_ref[0, :N_NODES, :] + q_ref[1, :N_NODES, :]
                  + b_ref[...])


def _tc_combine(q, b2):
    return pl.pallas_call(
        _comb_body,
        out_shape=jax.ShapeDtypeStruct((N_NODES, N_CLASSES), jnp.float32),
    )(q, b2)


def _make_sc_hop():
    mesh = plsc.VectorSubcoreMesh(core_axis_name="c", subcore_axis_name="s")
    scratch = (
        [pltpu.VMEM_SHARED((N_PAD, N_CLASSES), jnp.float32)] * 2
        + [pltpu.VMEM((2, CHUNK), jnp.int32)] * (2 * DEPTH)
        + [pltpu.VMEM((CHUNK, N_CLASSES), jnp.float32)] * DEPTH
        + [pltpu.SemaphoreType.DMA] * (3 + 4 * DEPTH)
    )

    @functools.partial(
        pl.kernel,
        out_type=jax.ShapeDtypeStruct((NC, N_PAD, N_CLASSES), jnp.float32),
        mesh=mesh,
        scratch_types=scratch,
        compiler_params=pltpu.CompilerParams(use_tc_tiling_on_sc=False),
    )
    def run(y_hbm, z_hbm, epk_hbm, out_hbm, y_sh, acc_sh, *rest):
        idx2 = [rest[:DEPTH], rest[DEPTH:2 * DEPTH]]
        rows = rest[2 * DEPTH:3 * DEPTH]
        sem_y, sem_z, sem_w = rest[3 * DEPTH:3 * DEPTH + 3]
        k = 3 * DEPTH + 3
        sem_i = [rest[k:k + DEPTH], rest[k + DEPTH:k + 2 * DEPTH]]
        sem_g = rest[k + 2 * DEPTH:k + 3 * DEPTH]
        sem_s = rest[k + 3 * DEPTH:]

        cid = lax.axis_index("c")
        sid = lax.axis_index("s")
        wid = sid * NC + cid
        r0 = sid * RPT

        dy = pltpu.async_copy(y_hbm.at[pl.ds(r0, RPT)],
                              y_sh.at[pl.ds(r0, RPT)], sem_y)
        dz = pltpu.async_copy(z_hbm.at[pl.ds(r0, RPT)],
                              acc_sh.at[pl.ds(r0, RPT)], sem_z)
        for j in range(DEPTH):
            pltpu.async_copy(epk_hbm.at[wid, j], idx2[0][j], sem_i[0][j])
        dy.wait()
        dz.wait()
        plsc.subcore_barrier()

        def one_group(g, par):
            ibuf = idx2[par]
            isem = sem_i[par]
            gd = []
            for j in range(DEPTH):
                pltpu.make_async_copy(epk_hbm.at[wid, 0], ibuf[j],
                                      isem[j]).wait()
                gd.append(pltpu.async_copy(y_sh.at[ibuf[j].at[0]], rows[j],
                                           sem_g[j]))

            @pl.when(g < NG - 1)
            def _():
                for j in range(DEPTH):
                    pltpu.async_copy(epk_hbm.at[wid, (g + 1) * DEPTH + j],
                                     idx2[1 - par][j], sem_i[1 - par][j])

            sd = []
            for j in range(DEPTH):
                gd[j].wait()
                sd.append(pltpu.async_copy(rows[j], acc_sh.at[ibuf[j].at[1]],
                                           sem_s[j], add=True))
            for j in range(DEPTH):
                sd[j].wait()

        def grp2(gp, carry):
            one_group(2 * gp, 0)
            one_group(2 * gp + 1, 1)
            return carry

        lax.fori_loop(0, NG // 2, grp2, 0)
        plsc.subcore_barrier()

        pltpu.async_copy(acc_sh.at[pl.ds(r0, RPT)],
                         out_hbm.at[cid, pl.ds(r0, RPT)], sem_w).wait()

    return run


_sc_hop = _make_sc_hop()


def kernel(feat, edge_index, W, b):
    fill = jnp.arange(PADE, dtype=jnp.int32) % (N_PAD - N_NODES)
    srcp = jnp.concatenate([edge_index[0], fill])
    dstp = jnp.concatenate([edge_index[1], N_NODES + fill])
    epk = jnp.stack([srcp.reshape(NW, NCH, CHUNK),
                     dstp.reshape(NW, NCH, CHUNK)], axis=2)
    z = jnp.zeros((N_PAD, N_CLASSES), jnp.float32)

    y0 = _tc_matmul(feat, W)
    p = _sc_hop(y0, z, epk)
    y1 = _tc_mid(p)
    q = _sc_hop(y1, z, epk)
    out = _tc_combine(q, jnp.broadcast_to(b, (1, N_CLASSES)))
    return out


# speedup vs baseline: 1.1667x; 1.1667x over previous
import functools

import jax
import jax.numpy as jnp
from jax import lax
from jax.experimental import pallas as pl
from jax.experimental.pallas import tpu as pltpu
from jax.experimental.pallas import tpu_sc as plsc

N_NODES = 10000
N_EDGES = 320000
D_FEAT = 128
N_CLASSES = 64

NC, NS = 2, 16
NW = NC * NS
CHUNK = 128
NCH = 80
E_PK = NW * NCH * CHUNK
PADE = E_PK - N_EDGES
N_PAD = 10240
RPT = N_PAD // NS
DEPTH = 4
NG = NCH // DEPTH


def _mm_body(feat_ref, w_ref, o_ref):
    o_ref[:N_NODES] = lax.dot_general(
        feat_ref[...], w_ref[...],
        (((1,), (1,)), ((), ())),
        preferred_element_type=jnp.float32,
    )
    o_ref[N_NODES:] = jnp.zeros((N_PAD - N_NODES, N_CLASSES), jnp.float32)


def _tc_matmul(feat, W):
    return pl.pallas_call(
        _mm_body,
        out_shape=jax.ShapeDtypeStruct((N_PAD, N_CLASSES), jnp.float32),
    )(feat, W)


def _mid_body(p_ref, o_ref):
    o_ref[...] = p_ref[0] + p_ref[1]


def _tc_mid(p):
    return pl.pallas_call(
        _mid_body,
        out_shape=jax.ShapeDtypeStruct((N_PAD, N_CLASSES), jnp.float32),
    )(p)


def _comb_body(q_ref, b_ref, o_ref):
    o_ref[...] = (q_ref[0, :N_NODES, :] + q_ref[1, :N_NODES, :]
                  + b_ref[...])


def _tc_combine(q, b2):
    return pl.pallas_call(
        _comb_body,
        out_shape=jax.ShapeDtypeStruct((N_NODES, N_CLASSES), jnp.float32),
    )(q, b2)


def _make_sc_hop():
    mesh = plsc.VectorSubcoreMesh(core_axis_name="c", subcore_axis_name="s")
    scratch = (
        [pltpu.VMEM_SHARED((N_PAD, N_CLASSES), jnp.float32)] * 2
        + [pltpu.VMEM((2, CHUNK), jnp.int32)] * (2 * DEPTH)
        + [pltpu.VMEM((CHUNK, N_CLASSES), jnp.float32)] * DEPTH
        + [pltpu.SemaphoreType.DMA] * (3 + 4 * DEPTH)
    )

    @functools.partial(
        pl.kernel,
        out_type=jax.ShapeDtypeStruct((NC, N_PAD, N_CLASSES), jnp.float32),
        mesh=mesh,
        scratch_types=scratch,
        compiler_params=pltpu.CompilerParams(use_tc_tiling_on_sc=False),
    )
    def run(y_hbm, z_hbm, epk_hbm, out_hbm, y_sh, acc_sh, *rest):
        idx2 = [rest[:DEPTH], rest[DEPTH:2 * DEPTH]]
        rows = rest[2 * DEPTH:3 * DEPTH]
        sem_y, sem_z, sem_w = rest[3 * DEPTH:3 * DEPTH + 3]
        k = 3 * DEPTH + 3
        sem_i = [rest[k:k + DEPTH], rest[k + DEPTH:k + 2 * DEPTH]]
        sem_g = rest[k + 2 * DEPTH:k + 3 * DEPTH]
        sem_s = rest[k + 3 * DEPTH:]

        cid = lax.axis_index("c")
        sid = lax.axis_index("s")
        wid = sid * NC + cid
        r0 = sid * RPT

        dy = pltpu.async_copy(y_hbm.at[pl.ds(r0, RPT)],
                              y_sh.at[pl.ds(r0, RPT)], sem_y)
        dz = pltpu.async_copy(z_hbm.at[pl.ds(r0, RPT)],
                              acc_sh.at[pl.ds(r0, RPT)], sem_z)
        for j in range(DEPTH):
            pltpu.async_copy(epk_hbm.at[wid, j], idx2[0][j], sem_i[0][j])
        dy.wait()
        dz.wait()
        plsc.subcore_barrier()

        def one_group(g, par):
            ibuf = idx2[par]
            isem = sem_i[par]
            gd = []
            for j in range(DEPTH):
                pltpu.make_async_copy(epk_hbm.at[wid, 0], ibuf[j],
                                      isem[j]).wait()
                gd.append(pltpu.async_copy(y_sh.at[ibuf[j].at[0]], rows[j],
                                           sem_g[j]))

            @pl.when(g < NG - 1)
            def _():
                for j in range(DEPTH):
                    pltpu.async_copy(epk_hbm.at[wid, (g + 1) * DEPTH + j],
                                     idx2[1 - par][j], sem_i[1 - par][j])

            sd = []
            for j in range(DEPTH):
                gd[j].wait()
                sd.append(pltpu.async_copy(rows[j], acc_sh.at[ibuf[j].at[1]],
                                           sem_s[j], add=True))
            for j in range(DEPTH):
                sd[j].wait()

        def grp2(gp, carry):
            one_group(2 * gp, 0)
            one_group(2 * gp + 1, 1)
            return carry

        lax.fori_loop(0, NG // 2, grp2, 0)
        plsc.subcore_barrier()

        pltpu.async_copy(acc_sh.at[pl.ds(r0, RPT)],
                         out_hbm.at[cid, pl.ds(r0, RPT)], sem_w).wait()

    return run


_sc_hop = _make_sc_hop()


def kernel(feat, edge_index, W, b):
    fill = jnp.arange(PADE, dtype=jnp.int32) % (N_PAD - N_NODES)
    srcp = jnp.concatenate([edge_index[0], fill])
    dstp = jnp.concatenate([edge_index[1], N_NODES + fill])
    epk = jnp.stack([srcp.reshape(NW, NCH, CHUNK),
                     dstp.reshape(NW, NCH, CHUNK)], axis=2)
    z = jnp.zeros((N_PAD, N_CLASSES), jnp.float32)

    y0 = _tc_matmul(feat, W)
    p = _sc_hop(y0, z, epk)
    y1 = _tc_mid(p)
    q = _sc_hop(y1, z, epk)
    out = _tc_combine(q, jnp.broadcast_to(b, (1, N_CLASSES)))
    return out
